# baseline (speedup 1.0000x reference)
import jax
import jax.numpy as jnp
from jax.experimental import pallas as pl
from jax.experimental.pallas import tpu as pltpu

_S = 196
_H = 224
_W = 224
_HB = 224
_NH = _H // _HB
_KB = _HB * _W


def _sp_mean_kernel(mask_ref, xa_ref, wf_ref, out_ref, acc_ref):
    k = pl.program_id(1)

    @pl.when(k == 0)
    def _init():
        acc_ref[...] = jnp.zeros_like(acc_ref)

    m = mask_ref[0].reshape(_S, _KB).astype(jnp.float32)
    x3 = xa_ref[0].reshape(3, _KB)
    xa = jnp.concatenate([x3, jnp.ones((1, _KB), jnp.float32)], 0)
    acc_ref[...] += jax.lax.dot_general(
        m, xa, (((1,), (1,)), ((), ())), preferred_element_type=jnp.float32)

    @pl.when(k == _NH - 1)
    def _finish():
        acc = acc_ref[...]
        counts = acc[:, 3:4]
        proj = jax.lax.dot_general(
            acc, wf_ref[...], (((1,), (0,)), ((), ())),
            preferred_element_type=jnp.float32)
        out_ref[0] = proj / counts


def kernel(X, masks, W, b):
    B = X.shape[0]
    Wf = jnp.concatenate([W.T, b[None, :]], axis=0)

    out = pl.pallas_call(
        _sp_mean_kernel,
        grid=(B, _NH),
        in_specs=[
            pl.BlockSpec((1, _S, _HB, _W), lambda bi, ki: (bi, 0, ki, 0)),
            pl.BlockSpec((1, 3, _HB, _W), lambda bi, ki: (bi, 0, ki, 0)),
            pl.BlockSpec((4, 56), lambda bi, ki: (0, 0)),
        ],
        out_specs=pl.BlockSpec((1, _S, 56), lambda bi, ki: (bi, 0, 0)),
        out_shape=jax.ShapeDtypeStruct((B, _S, 56), jnp.float32),
        scratch_shapes=[pltpu.VMEM((_S, 4), jnp.float32)],
    )(masks.view(jnp.uint8), X, Wf)
    return out
